# baseline (speedup 1.0000x reference)
import functools

import jax
import jax.numpy as jnp
from jax.experimental import pallas as pl
from jax.experimental.pallas import tpu as pltpu

HIDDEN = 1024
N_EXPERTS = 8
INTER = 512
SHARED_DIM = 512
TT = 256


def _dotT(a, b):
    return jax.lax.dot_general(a, b, (((1,), (1,)), ((), ())),
                               preferred_element_type=jnp.float32)


def _routed_body(x_ref, gate_ref, w1_ref, w2_ref, out_ref, comb_ref):
    e = pl.program_id(1)

    @pl.when(e == 0)
    def _():
        x = x_ref[...]
        logits = _dotT(x, gate_ref[...])
        m = jnp.max(logits, axis=-1, keepdims=True)
        ex = jnp.exp(logits - m)
        p = ex / jnp.sum(ex, axis=-1, keepdims=True)
        idx = jax.lax.broadcasted_iota(jnp.int32, p.shape, 1)
        m1 = jnp.max(p, axis=-1, keepdims=True)
        i1 = jnp.min(jnp.where(p == m1, idx, N_EXPERTS), axis=-1, keepdims=True)
        oh1 = (idx == i1)
        p2 = jnp.where(oh1, -jnp.inf, p)
        m2 = jnp.max(p2, axis=-1, keepdims=True)
        i2 = jnp.min(jnp.where(p2 == m2, idx, N_EXPERTS), axis=-1, keepdims=True)
        oh2 = (idx == i2)
        denom = m1 + m2
        comb = jnp.where(oh1, m1 / denom, 0.0) + jnp.where(oh2, m2 / denom, 0.0)
        comb_ref[...] = comb

    x = x_ref[...]
    h = _dotT(x, w1_ref[0])
    h = h * jax.nn.sigmoid(h)
    y = _dotT(h, w2_ref[0])
    oh_e = (jax.lax.broadcasted_iota(jnp.int32, (1, N_EXPERTS), 1) == e)
    w_col = jnp.sum(jnp.where(oh_e, comb_ref[...], 0.0), axis=-1, keepdims=True)
    contrib = w_col * y

    @pl.when(e == 0)
    def _():
        out_ref[...] = contrib

    @pl.when(e != 0)
    def _():
        out_ref[...] += contrib


def _shared_body(x_ref, wgu_ref, wd_ref, routed_ref, out_ref):
    x = x_ref[...]
    gu = _dotT(x, wgu_ref[...])
    gate = gu[:, :SHARED_DIM]
    up = gu[:, SHARED_DIM:]
    s = gate * jax.nn.sigmoid(gate) * up
    y = _dotT(s, wd_ref[...])
    out_ref[...] = y + routed_ref[...]


def kernel(hidden_states, gate_w, w1, w2, shared_gate_up_w, shared_down_w):
    orig_shape = hidden_states.shape
    T = orig_shape[0] * orig_shape[1]
    x2d = hidden_states.reshape(T, HIDDEN)
    n_tiles = T // TT

    routed = pl.pallas_call(
        _routed_body,
        grid=(n_tiles, N_EXPERTS),
        in_specs=[
            pl.BlockSpec((TT, HIDDEN), lambda t, e: (t, 0)),
            pl.BlockSpec((N_EXPERTS, HIDDEN), lambda t, e: (0, 0)),
            pl.BlockSpec((1, INTER, HIDDEN), lambda t, e: (e, 0, 0)),
            pl.BlockSpec((1, HIDDEN, INTER), lambda t, e: (e, 0, 0)),
        ],
        out_specs=pl.BlockSpec((TT, HIDDEN), lambda t, e: (t, 0)),
        out_shape=jax.ShapeDtypeStruct((T, HIDDEN), jnp.float32),
        scratch_shapes=[pltpu.VMEM((TT, N_EXPERTS), jnp.float32)],
    )(x2d, gate_w, w1, w2)

    out = pl.pallas_call(
        _shared_body,
        grid=(n_tiles,),
        in_specs=[
            pl.BlockSpec((TT, HIDDEN), lambda t: (t, 0)),
            pl.BlockSpec((2 * SHARED_DIM, HIDDEN), lambda t: (0, 0)),
            pl.BlockSpec((HIDDEN, SHARED_DIM), lambda t: (0, 0)),
            pl.BlockSpec((TT, HIDDEN), lambda t: (t, 0)),
        ],
        out_specs=pl.BlockSpec((TT, HIDDEN), lambda t: (t, 0)),
        out_shape=jax.ShapeDtypeStruct((T, HIDDEN), jnp.float32),
    )(x2d, shared_gate_up_w, shared_down_w, routed)

    return out.reshape(orig_shape)


# speedup vs baseline: 1.0211x; 1.0211x over previous
import functools

import jax
import jax.numpy as jnp
from jax.experimental import pallas as pl
from jax.experimental.pallas import tpu as pltpu

HIDDEN = 1024
N_EXPERTS = 8
INTER = 512
SHARED_DIM = 512
TT = 256


def _dotT(a, b):
    return jax.lax.dot_general(a, b, (((1,), (1,)), ((), ())),
                               preferred_element_type=jnp.float32)


def _routed_body(x_ref, gate_ref, w1_ref, w2_ref, out_ref, comb_ref):
    e = pl.program_id(1)

    @pl.when(e == 0)
    def _():
        x = x_ref[...]
        logits = _dotT(x, gate_ref[...])
        m = jnp.max(logits, axis=-1, keepdims=True)
        ex = jnp.exp(logits - m)
        p = ex / jnp.sum(ex, axis=-1, keepdims=True)
        idx = jax.lax.broadcasted_iota(jnp.int32, p.shape, 1)
        m1 = jnp.max(p, axis=-1, keepdims=True)
        i1 = jnp.min(jnp.where(p == m1, idx, N_EXPERTS), axis=-1, keepdims=True)
        oh1 = (idx == i1)
        p2 = jnp.where(oh1, -jnp.inf, p)
        m2 = jnp.max(p2, axis=-1, keepdims=True)
        i2 = jnp.min(jnp.where(p2 == m2, idx, N_EXPERTS), axis=-1, keepdims=True)
        oh2 = (idx == i2)
        denom = m1 + m2
        comb = jnp.where(oh1, m1 / denom, 0.0) + jnp.where(oh2, m2 / denom, 0.0)
        comb_ref[...] = comb

    x = x_ref[...].astype(jnp.bfloat16)
    h = _dotT(x, w1_ref[0])
    h = h * jax.nn.sigmoid(h)
    y = _dotT(h.astype(jnp.bfloat16), w2_ref[0])
    oh_e = (jax.lax.broadcasted_iota(jnp.int32, (1, N_EXPERTS), 1) == e)
    w_col = jnp.sum(jnp.where(oh_e, comb_ref[...], 0.0), axis=-1, keepdims=True)
    contrib = w_col * y

    @pl.when(e == 0)
    def _():
        out_ref[...] = contrib

    @pl.when(e != 0)
    def _():
        out_ref[...] += contrib


def _shared_body(x_ref, wgu_ref, wd_ref, routed_ref, out_ref):
    x = x_ref[...].astype(jnp.bfloat16)
    gu = _dotT(x, wgu_ref[...])
    gate = gu[:, :SHARED_DIM]
    up = gu[:, SHARED_DIM:]
    s = gate * jax.nn.sigmoid(gate) * up
    y = _dotT(s.astype(jnp.bfloat16), wd_ref[...])
    out_ref[...] = y + routed_ref[...]


def kernel(hidden_states, gate_w, w1, w2, shared_gate_up_w, shared_down_w):
    orig_shape = hidden_states.shape
    T = orig_shape[0] * orig_shape[1]
    x2d = hidden_states.reshape(T, HIDDEN)
    n_tiles = T // TT
    w1 = w1.astype(jnp.bfloat16)
    w2 = w2.astype(jnp.bfloat16)
    shared_gate_up_w = shared_gate_up_w.astype(jnp.bfloat16)
    shared_down_w = shared_down_w.astype(jnp.bfloat16)

    routed = pl.pallas_call(
        _routed_body,
        grid=(n_tiles, N_EXPERTS),
        in_specs=[
            pl.BlockSpec((TT, HIDDEN), lambda t, e: (t, 0)),
            pl.BlockSpec((N_EXPERTS, HIDDEN), lambda t, e: (0, 0)),
            pl.BlockSpec((1, INTER, HIDDEN), lambda t, e: (e, 0, 0)),
            pl.BlockSpec((1, HIDDEN, INTER), lambda t, e: (e, 0, 0)),
        ],
        out_specs=pl.BlockSpec((TT, HIDDEN), lambda t, e: (t, 0)),
        out_shape=jax.ShapeDtypeStruct((T, HIDDEN), jnp.float32),
        scratch_shapes=[pltpu.VMEM((TT, N_EXPERTS), jnp.float32)],
    )(x2d, gate_w, w1, w2)

    out = pl.pallas_call(
        _shared_body,
        grid=(n_tiles,),
        in_specs=[
            pl.BlockSpec((TT, HIDDEN), lambda t: (t, 0)),
            pl.BlockSpec((2 * SHARED_DIM, HIDDEN), lambda t: (0, 0)),
            pl.BlockSpec((HIDDEN, SHARED_DIM), lambda t: (0, 0)),
            pl.BlockSpec((TT, HIDDEN), lambda t: (t, 0)),
        ],
        out_specs=pl.BlockSpec((TT, HIDDEN), lambda t: (t, 0)),
        out_shape=jax.ShapeDtypeStruct((T, HIDDEN), jnp.float32),
    )(x2d, shared_gate_up_w, shared_down_w, routed)

    return out.reshape(orig_shape)


# speedup vs baseline: 1.3419x; 1.3141x over previous
import functools

import jax
import jax.numpy as jnp
from jax.experimental import pallas as pl
from jax.experimental.pallas import tpu as pltpu

HIDDEN = 1024
N_EXPERTS = 8
INTER = 512
SHARED_DIM = 512


def _dotT(a, b):
    return jax.lax.dot_general(a, b, (((1,), (1,)), ((), ())),
                               preferred_element_type=jnp.float32)


def _router_body(x_ref, gate_ref, comb_ref):
    logits = _dotT(x_ref[...], gate_ref[...])
    m = jnp.max(logits, axis=-1, keepdims=True)
    ex = jnp.exp(logits - m)
    p = ex / jnp.sum(ex, axis=-1, keepdims=True)
    idx = jax.lax.broadcasted_iota(jnp.int32, p.shape, 1)
    m1 = jnp.max(p, axis=-1, keepdims=True)
    i1 = jnp.min(jnp.where(p == m1, idx, N_EXPERTS), axis=-1, keepdims=True)
    oh1 = (idx == i1)
    p2 = jnp.where(oh1, -jnp.inf, p)
    m2 = jnp.max(p2, axis=-1, keepdims=True)
    i2 = jnp.min(jnp.where(p2 == m2, idx, N_EXPERTS), axis=-1, keepdims=True)
    oh2 = (idx == i2)
    denom = m1 + m2
    comb_ref[...] = (jnp.where(oh1, m1 / denom, 0.0)
                     + jnp.where(oh2, m2 / denom, 0.0))


def _experts_body(x_ref, w1_ref, w2_ref, comb_ref, out_ref):
    e = pl.program_id(0)
    x = x_ref[...]
    h = _dotT(x, w1_ref[0])
    h = h * jax.nn.sigmoid(h)
    y = _dotT(h.astype(jnp.bfloat16), w2_ref[0])
    oh_e = (jax.lax.broadcasted_iota(jnp.int32, (1, N_EXPERTS), 1) == e)
    w_col = jnp.sum(jnp.where(oh_e, comb_ref[...], 0.0), axis=-1, keepdims=True)
    contrib = w_col * y

    @pl.when(e == 0)
    def _():
        out_ref[...] = contrib

    @pl.when(e != 0)
    def _():
        out_ref[...] += contrib


def _shared_body(x_ref, wgu_ref, wd_ref, routed_ref, out_ref):
    x = x_ref[...]
    gu = _dotT(x, wgu_ref[...])
    gate = gu[:, :SHARED_DIM]
    up = gu[:, SHARED_DIM:]
    s = gate * jax.nn.sigmoid(gate) * up
    y = _dotT(s.astype(jnp.bfloat16), wd_ref[...])
    out_ref[...] = y + routed_ref[...]


def kernel(hidden_states, gate_w, w1, w2, shared_gate_up_w, shared_down_w):
    orig_shape = hidden_states.shape
    T = orig_shape[0] * orig_shape[1]
    x2d = hidden_states.reshape(T, HIDDEN)
    x_bf = x2d.astype(jnp.bfloat16)
    w1 = w1.astype(jnp.bfloat16)
    w2 = w2.astype(jnp.bfloat16)
    shared_gate_up_w = shared_gate_up_w.astype(jnp.bfloat16)
    shared_down_w = shared_down_w.astype(jnp.bfloat16)

    comb = pl.pallas_call(
        _router_body,
        grid=(1,),
        in_specs=[
            pl.BlockSpec((T, HIDDEN), lambda i: (0, 0)),
            pl.BlockSpec((N_EXPERTS, HIDDEN), lambda i: (0, 0)),
        ],
        out_specs=pl.BlockSpec((T, N_EXPERTS), lambda i: (0, 0)),
        out_shape=jax.ShapeDtypeStruct((T, N_EXPERTS), jnp.float32),
    )(x2d, gate_w)

    routed = pl.pallas_call(
        _experts_body,
        grid=(N_EXPERTS,),
        in_specs=[
            pl.BlockSpec((T, HIDDEN), lambda e: (0, 0)),
            pl.BlockSpec((1, INTER, HIDDEN), lambda e: (e, 0, 0)),
            pl.BlockSpec((1, HIDDEN, INTER), lambda e: (e, 0, 0)),
            pl.BlockSpec((T, N_EXPERTS), lambda e: (0, 0)),
        ],
        out_specs=pl.BlockSpec((T, HIDDEN), lambda e: (0, 0)),
        out_shape=jax.ShapeDtypeStruct((T, HIDDEN), jnp.float32),
    )(x_bf, w1, w2, comb)

    out = pl.pallas_call(
        _shared_body,
        grid=(1,),
        in_specs=[
            pl.BlockSpec((T, HIDDEN), lambda i: (0, 0)),
            pl.BlockSpec((2 * SHARED_DIM, HIDDEN), lambda i: (0, 0)),
            pl.BlockSpec((HIDDEN, SHARED_DIM), lambda i: (0, 0)),
            pl.BlockSpec((T, HIDDEN), lambda i: (0, 0)),
        ],
        out_specs=pl.BlockSpec((T, HIDDEN), lambda i: (0, 0)),
        out_shape=jax.ShapeDtypeStruct((T, HIDDEN), jnp.float32),
    )(x_bf, shared_gate_up_w, shared_down_w, routed)

    return out.reshape(orig_shape)


# speedup vs baseline: 1.7493x; 1.3036x over previous
import functools

import jax
import jax.numpy as jnp
from jax.experimental import pallas as pl
from jax.experimental.pallas import tpu as pltpu

HIDDEN = 1024
N_EXPERTS = 8
INTER = 512
SHARED_DIM = 512


def _dotT(a, b):
    return jax.lax.dot_general(a, b, (((1,), (1,)), ((), ())),
                               preferred_element_type=jnp.float32)


def _router_body(x_ref, gate_ref, comb_ref, xbf_ref):
    x = x_ref[...]
    xbf_ref[...] = x.astype(jnp.bfloat16)
    logits = _dotT(x, gate_ref[...])
    m = jnp.max(logits, axis=-1, keepdims=True)
    ex = jnp.exp(logits - m)
    p = ex / jnp.sum(ex, axis=-1, keepdims=True)
    idx = jax.lax.broadcasted_iota(jnp.int32, p.shape, 1)
    m1 = jnp.max(p, axis=-1, keepdims=True)
    i1 = jnp.min(jnp.where(p == m1, idx, N_EXPERTS), axis=-1, keepdims=True)
    oh1 = (idx == i1)
    p2 = jnp.where(oh1, -jnp.inf, p)
    m2 = jnp.max(p2, axis=-1, keepdims=True)
    i2 = jnp.min(jnp.where(p2 == m2, idx, N_EXPERTS), axis=-1, keepdims=True)
    oh2 = (idx == i2)
    denom = m1 + m2
    comb_ref[...] = (jnp.where(oh1, m1 / denom, 0.0)
                     + jnp.where(oh2, m2 / denom, 0.0))


def _experts_body(x_ref, w1_ref, w2_ref, comb_ref, out_ref):
    e = pl.program_id(0)
    x = x_ref[...]
    w1 = w1_ref[0].astype(jnp.bfloat16)
    h = _dotT(x, w1)
    h = h * jax.nn.sigmoid(h)
    w2 = w2_ref[0].astype(jnp.bfloat16)
    y = _dotT(h.astype(jnp.bfloat16), w2)
    oh_e = (jax.lax.broadcasted_iota(jnp.int32, (1, N_EXPERTS), 1) == e)
    w_col = jnp.sum(jnp.where(oh_e, comb_ref[...], 0.0), axis=-1, keepdims=True)
    contrib = w_col * y

    @pl.when(e == 0)
    def _():
        out_ref[...] = contrib

    @pl.when(e != 0)
    def _():
        out_ref[...] += contrib


def _shared_body(x_ref, wgu_ref, wd_ref, routed_ref, out_ref):
    x = x_ref[...]
    gu = _dotT(x, wgu_ref[...].astype(jnp.bfloat16))
    gate = gu[:, :SHARED_DIM]
    up = gu[:, SHARED_DIM:]
    s = gate * jax.nn.sigmoid(gate) * up
    y = _dotT(s.astype(jnp.bfloat16), wd_ref[...].astype(jnp.bfloat16))
    out_ref[...] = y + routed_ref[...]


def kernel(hidden_states, gate_w, w1, w2, shared_gate_up_w, shared_down_w):
    orig_shape = hidden_states.shape
    T = orig_shape[0] * orig_shape[1]
    x2d = hidden_states.reshape(T, HIDDEN)

    comb, x_bf = pl.pallas_call(
        _router_body,
        grid=(1,),
        in_specs=[
            pl.BlockSpec((T, HIDDEN), lambda i: (0, 0)),
            pl.BlockSpec((N_EXPERTS, HIDDEN), lambda i: (0, 0)),
        ],
        out_specs=[
            pl.BlockSpec((T, N_EXPERTS), lambda i: (0, 0)),
            pl.BlockSpec((T, HIDDEN), lambda i: (0, 0)),
        ],
        out_shape=[
            jax.ShapeDtypeStruct((T, N_EXPERTS), jnp.float32),
            jax.ShapeDtypeStruct((T, HIDDEN), jnp.bfloat16),
        ],
    )(x2d, gate_w)

    routed = pl.pallas_call(
        _experts_body,
        grid=(N_EXPERTS,),
        in_specs=[
            pl.BlockSpec((T, HIDDEN), lambda e: (0, 0)),
            pl.BlockSpec((1, INTER, HIDDEN), lambda e: (e, 0, 0)),
            pl.BlockSpec((1, HIDDEN, INTER), lambda e: (e, 0, 0)),
            pl.BlockSpec((T, N_EXPERTS), lambda e: (0, 0)),
        ],
        out_specs=pl.BlockSpec((T, HIDDEN), lambda e: (0, 0)),
        out_shape=jax.ShapeDtypeStruct((T, HIDDEN), jnp.float32),
    )(x_bf, w1, w2, comb)

    out = pl.pallas_call(
        _shared_body,
        grid=(1,),
        in_specs=[
            pl.BlockSpec((T, HIDDEN), lambda i: (0, 0)),
            pl.BlockSpec((2 * SHARED_DIM, HIDDEN), lambda i: (0, 0)),
            pl.BlockSpec((HIDDEN, SHARED_DIM), lambda i: (0, 0)),
            pl.BlockSpec((T, HIDDEN), lambda i: (0, 0)),
        ],
        out_specs=pl.BlockSpec((T, HIDDEN), lambda i: (0, 0)),
        out_shape=jax.ShapeDtypeStruct((T, HIDDEN), jnp.float32),
    )(x_bf, shared_gate_up_w, shared_down_w, routed)

    return out.reshape(orig_shape)


# speedup vs baseline: 2.0413x; 1.1670x over previous
import functools

import jax
import jax.numpy as jnp
from jax.experimental import pallas as pl
from jax.experimental.pallas import tpu as pltpu

HIDDEN = 1024
N_EXPERTS = 8
INTER = 512
SHARED_DIM = 512


def _dotT(a, b):
    return jax.lax.dot_general(a, b, (((1,), (1,)), ((), ())),
                               preferred_element_type=jnp.float32)


def _moe_body(x_ref, gate_ref, w1_ref, w2_ref, wgu_ref, wd_ref,
              out_ref, xbf_ref, comb_ref):
    e = pl.program_id(0)

    @pl.when(e == 0)
    def _():
        x = x_ref[...]
        xbf_ref[...] = x.astype(jnp.bfloat16)
        logits = _dotT(x, gate_ref[...])
        m = jnp.max(logits, axis=-1, keepdims=True)
        ex = jnp.exp(logits - m)
        p = ex / jnp.sum(ex, axis=-1, keepdims=True)
        idx = jax.lax.broadcasted_iota(jnp.int32, p.shape, 1)
        m1 = jnp.max(p, axis=-1, keepdims=True)
        i1 = jnp.min(jnp.where(p == m1, idx, N_EXPERTS), axis=-1, keepdims=True)
        oh1 = (idx == i1)
        p2 = jnp.where(oh1, -jnp.inf, p)
        m2 = jnp.max(p2, axis=-1, keepdims=True)
        i2 = jnp.min(jnp.where(p2 == m2, idx, N_EXPERTS), axis=-1, keepdims=True)
        oh2 = (idx == i2)
        denom = m1 + m2
        comb_ref[...] = (jnp.where(oh1, m1 / denom, 0.0)
                         + jnp.where(oh2, m2 / denom, 0.0))

    @pl.when(e < N_EXPERTS)
    def _():
        x = xbf_ref[...]
        h = _dotT(x, w1_ref[0].astype(jnp.bfloat16))
        h = h * jax.nn.sigmoid(h)
        y = _dotT(h.astype(jnp.bfloat16),
                  w2_ref[0].astype(jnp.bfloat16))
        oh_e = (jax.lax.broadcasted_iota(jnp.int32, (1, N_EXPERTS), 1) == e)
        w_col = jnp.sum(jnp.where(oh_e, comb_ref[...], 0.0),
                        axis=-1, keepdims=True)
        contrib = w_col * y

        @pl.when(e == 0)
        def _():
            out_ref[...] = contrib

        @pl.when(e != 0)
        def _():
            out_ref[...] += contrib

    @pl.when(e == N_EXPERTS)
    def _():
        x = xbf_ref[...]
        gu = _dotT(x, wgu_ref[...].astype(jnp.bfloat16))
        gate = gu[:, :SHARED_DIM]
        up = gu[:, SHARED_DIM:]
        s = gate * jax.nn.sigmoid(gate) * up
        y = _dotT(s.astype(jnp.bfloat16), wd_ref[...].astype(jnp.bfloat16))
        out_ref[...] += y


def kernel(hidden_states, gate_w, w1, w2, shared_gate_up_w, shared_down_w):
    orig_shape = hidden_states.shape
    T = orig_shape[0] * orig_shape[1]
    x2d = hidden_states.reshape(T, HIDDEN)

    out = pl.pallas_call(
        _moe_body,
        grid=(N_EXPERTS + 1,),
        in_specs=[
            pl.BlockSpec((T, HIDDEN), lambda e: (0, 0)),
            pl.BlockSpec((N_EXPERTS, HIDDEN), lambda e: (0, 0)),
            pl.BlockSpec((1, INTER, HIDDEN),
                         lambda e: (jnp.minimum(e, N_EXPERTS - 1), 0, 0)),
            pl.BlockSpec((1, HIDDEN, INTER),
                         lambda e: (jnp.minimum(e, N_EXPERTS - 1), 0, 0)),
            pl.BlockSpec((2 * SHARED_DIM, HIDDEN), lambda e: (0, 0)),
            pl.BlockSpec((HIDDEN, SHARED_DIM), lambda e: (0, 0)),
        ],
        out_specs=pl.BlockSpec((T, HIDDEN), lambda e: (0, 0)),
        out_shape=jax.ShapeDtypeStruct((T, HIDDEN), jnp.float32),
        scratch_shapes=[
            pltpu.VMEM((T, HIDDEN), jnp.bfloat16),
            pltpu.VMEM((T, N_EXPERTS), jnp.float32),
        ],
    )(x2d, gate_w, w1, w2, shared_gate_up_w, shared_down_w)

    return out.reshape(orig_shape)
